# baseline (speedup 1.0000x reference)
import jax
import jax.numpy as jnp
from jax import lax
from jax.experimental import pallas as pl
from jax.experimental.pallas import tpu as pltpu
from jax.experimental.pallas import tpu_sc as plsc

N = 10000
E = 320000
D = 128
H = 128
C = 64

NC = 2
NS = 16
NW = NC * NS
EPW = E // NW
K = 80
NCHUNK = EPW // K
RPT = 624
RTAIL = N - NS * RPT

def _make_sc_agg(width, with_deg):
    mesh = plsc.VectorSubcoreMesh(core_axis_name="c", subcore_axis_name="s",
                                  num_cores=NC, num_subcores=NS)
    out_type = [jax.ShapeDtypeStruct((NC, N, width), jnp.float32)]
    if with_deg:
        out_type.append(jax.ShapeDtypeStruct((NC, N, 16), jnp.float32))

    NB = 2 if width > 64 else 5
    scratch = [
        pltpu.VMEM((EPW,), jnp.int32),
        [pltpu.VMEM((K,), jnp.int32) for _ in range(NB)],
        [pltpu.VMEM((K, 16), jnp.float32) for _ in range(NB)],
        [pltpu.VMEM((K, width), jnp.float32) for _ in range(NB)],
        pltpu.VMEM_SHARED((N, width), jnp.float32),
    ]
    if with_deg:
        scratch.append(pltpu.VMEM_SHARED((N, 16), jnp.float32))
    scratch.append([pltpu.SemaphoreType.DMA for _ in range(NB)])
    scratch.append([pltpu.SemaphoreType.DMA for _ in range(NB)])

    def body(*refs):
        if with_deg:
            (x_hbm, a_hbm, w_hbm, out_hbm, deg_hbm,
             idx_s, idx_d, wvr, rows, acc, dacc, gsem, isem) = refs
        else:
            (x_hbm, a_hbm, w_hbm, out_hbm,
             idx_s, idx_d, wvr, rows, acc, gsem, isem) = refs
        c = lax.axis_index("c")
        s = lax.axis_index("s")
        wid = s * NC + c
        r0 = s * RPT
        e0 = wid * EPW

        pltpu.sync_copy(a_hbm.at[0, pl.ds(e0, EPW)], idx_s)

        zv = jnp.zeros((16,), jnp.float32)

        def zrow(r, c2):
            for j in range(width // 16):
                rows[0][r, pl.ds(j * 16, 16)] = zv
            wvr[0][r, pl.ds(0, 16)] = zv
            return c2
        lax.fori_loop(0, K, zrow, 0)

        for q in range(RPT // K):
            pltpu.sync_copy(rows[0], acc.at[pl.ds(r0 + q * K, K)])
            if with_deg:
                pltpu.sync_copy(wvr[0], dacc.at[pl.ds(r0 + q * K, K)])
        rem = RPT % K
        if rem:
            pltpu.sync_copy(rows[0].at[pl.ds(0, rem)],
                            acc.at[pl.ds(r0 + (RPT // K) * K, rem)])
            if with_deg:
                pltpu.sync_copy(wvr[0].at[pl.ds(0, rem)],
                                dacc.at[pl.ds(r0 + (RPT // K) * K, rem)])

        @pl.when(s == 0)
        def _():
            t0 = NS * RPT
            pltpu.sync_copy(rows[0].at[pl.ds(0, RTAIL)],
                            acc.at[pl.ds(t0, RTAIL)])
            if with_deg:
                pltpu.sync_copy(wvr[0].at[pl.ds(0, RTAIL)],
                                dacc.at[pl.ds(t0, RTAIL)])
        plsc.subcore_barrier()

        def issue(t, b):
            base = e0 + t * K
            pltpu.async_copy(a_hbm.at[1, pl.ds(base, K)], idx_d[b], isem[b])
            pltpu.async_copy(w_hbm.at[pl.ds(base, K)], wvr[b], isem[b])
            pltpu.async_copy(x_hbm.at[idx_s.at[pl.ds(t * K, K)]], rows[b],
                             gsem[b])

        for b in range(NB):
            issue(b, b)

        def process(t, b, do_issue):
            pltpu.make_async_copy(a_hbm.at[1, pl.ds(e0, K)], idx_d[b],
                                  isem[b]).wait()
            pltpu.make_async_copy(w_hbm.at[pl.ds(e0, K)], wvr[b],
                                  isem[b]).wait()
            pltpu.make_async_copy(x_hbm.at[pl.ds(0, K)], rows[b],
                                  gsem[b]).wait()

            def grp(g, c2):
                for i in range(16):
                    e = g * 16 + i
                    ws = wvr[b][e]
                    for j in range(width // 16):
                        rows[b][e, pl.ds(j * 16, 16)] = (
                            rows[b][e, pl.ds(j * 16, 16)] * ws)
                return c2
            lax.fori_loop(0, K // 16, grp, 0)

            pltpu.sync_copy(rows[b], acc.at[idx_d[b]], add=True)
            if with_deg:
                pltpu.sync_copy(wvr[b], dacc.at[idx_d[b]], add=True)

            if do_issue:
                @pl.when(t + NB < NCHUNK)
                def _():
                    issue(t + NB, b)

        def ring(t2, carry):
            for b in range(NB):
                process(t2 * NB + b, b, True)
            return carry
        lax.fori_loop(0, NCHUNK // NB, ring, 0)
        for r in range(NCHUNK % NB):
            process((NCHUNK // NB) * NB + r, r, False)

        plsc.subcore_barrier()
        pltpu.sync_copy(acc.at[pl.ds(r0, RPT)], out_hbm.at[c, pl.ds(r0, RPT)])
        if with_deg:
            pltpu.sync_copy(dacc.at[pl.ds(r0, RPT)],
                            deg_hbm.at[c, pl.ds(r0, RPT)])

        @pl.when(s == 0)
        def _():
            t0 = NS * RPT
            pltpu.sync_copy(acc.at[pl.ds(t0, RTAIL)],
                            out_hbm.at[c, pl.ds(t0, RTAIL)])
            if with_deg:
                pltpu.sync_copy(dacc.at[pl.ds(t0, RTAIL)],
                                deg_hbm.at[c, pl.ds(t0, RTAIL)])

    return pl.kernel(body, out_type=tuple(out_type), mesh=mesh,
                     scratch_types=tuple(scratch),
                     compiler_params=pltpu.CompilerParams(
                         use_tc_tiling_on_sc=False,
                         needs_layout_passes=False))


_sc_agg_l1 = _make_sc_agg(D, True)
_sc_agg_l2 = _make_sc_agg(C, False)


BROWS = 1000


def _tc1_body(p, d, x, wl1, wr1, b1, wl2, wr2, y2, r2):
    deg = d[0][:, 0:1] + d[1][:, 0:1]
    agg = (p[0] + p[1]) / jnp.clip(deg, 1e-12, None)
    out = (jnp.dot(agg, wl1[...], preferred_element_type=jnp.float32)
           + jnp.dot(x[...], wr1[...], preferred_element_type=jnp.float32)
           + b1[...])
    nrm = jnp.sqrt(jnp.sum(out * out, axis=1, keepdims=True))
    h = jnp.maximum(out / jnp.clip(nrm, 1e-12, None), 0.0)
    y2[...] = jnp.dot(h, wl2[...], preferred_element_type=jnp.float32)
    r2[...] = jnp.dot(h, wr2[...], preferred_element_type=jnp.float32)


_tc1 = pl.pallas_call(
    _tc1_body,
    grid=(N // BROWS,),
    in_specs=[
        pl.BlockSpec((2, BROWS, D), lambda i: (0, i, 0)),
        pl.BlockSpec((2, BROWS, 16), lambda i: (0, i, 0)),
        pl.BlockSpec((BROWS, D), lambda i: (i, 0)),
        pl.BlockSpec((D, H), lambda i: (0, 0)),
        pl.BlockSpec((D, H), lambda i: (0, 0)),
        pl.BlockSpec((1, H), lambda i: (0, 0)),
        pl.BlockSpec((H, C), lambda i: (0, 0)),
        pl.BlockSpec((H, C), lambda i: (0, 0)),
    ],
    out_specs=[
        pl.BlockSpec((BROWS, C), lambda i: (i, 0)),
        pl.BlockSpec((BROWS, C), lambda i: (i, 0)),
    ],
    out_shape=[
        jax.ShapeDtypeStruct((N, C), jnp.float32),
        jax.ShapeDtypeStruct((N, C), jnp.float32),
    ],
)


def _tc2_body(q, d, r2, b2, out):
    deg = d[0][:, 0:1] + d[1][:, 0:1]
    z = (q[0] + q[1]) / jnp.clip(deg, 1e-12, None) + r2[...] + b2[...]
    m = jnp.max(z, axis=1, keepdims=True)
    zs = z - m
    out[...] = zs - jnp.log(jnp.sum(jnp.exp(zs), axis=1, keepdims=True))


_tc2 = pl.pallas_call(
    _tc2_body,
    grid=(N // BROWS,),
    in_specs=[
        pl.BlockSpec((2, BROWS, C), lambda i: (0, i, 0)),
        pl.BlockSpec((2, BROWS, 16), lambda i: (0, i, 0)),
        pl.BlockSpec((BROWS, C), lambda i: (i, 0)),
        pl.BlockSpec((1, C), lambda i: (0, 0)),
    ],
    out_specs=pl.BlockSpec((BROWS, C), lambda i: (i, 0)),
    out_shape=jax.ShapeDtypeStruct((N, C), jnp.float32),
)


def kernel(X, A, W, Wl1, Wr1, b1, Wl2, Wr2, b2):
    wrep = jnp.broadcast_to(W[:, None], (E, 16))
    P, DEG = _sc_agg_l1(X, A, wrep)
    Y2, R2 = _tc1(P, DEG, X, Wl1.T, Wr1.T, b1[None, :], Wl2.T, Wr2.T)
    (Q,) = _sc_agg_l2(Y2, A, wrep)
    return _tc2(Q, DEG, R2, b2[None, :])


# speedup vs baseline: 1.6308x; 1.6308x over previous
import jax
import jax.numpy as jnp
from jax import lax
from jax.experimental import pallas as pl
from jax.experimental.pallas import tpu as pltpu
from jax.experimental.pallas import tpu_sc as plsc

N = 10000
E = 320000
D = 128
H = 128
C = 64

NC = 2
NS = 16
NW = NC * NS
EPW = E // NW
K = 80
NCHUNK = EPW // K
RPT = 624
RTAIL = N - NS * RPT

def _make_sc_agg(width, with_deg):
    mesh = plsc.VectorSubcoreMesh(core_axis_name="c", subcore_axis_name="s",
                                  num_cores=NC, num_subcores=NS)
    out_type = [jax.ShapeDtypeStruct((NC, N, width), jnp.float32)]
    if with_deg:
        out_type.append(jax.ShapeDtypeStruct((NC, N, 16), jnp.float32))

    NB = 2 if width > 64 else 5
    scratch = [
        pltpu.VMEM((EPW,), jnp.int32),
        [pltpu.VMEM((K,), jnp.int32) for _ in range(NB)],
        [pltpu.VMEM((K,), jnp.float32) for _ in range(NB)],
        [pltpu.VMEM((K, width), jnp.float32) for _ in range(NB)],
        pltpu.VMEM((K, 16), jnp.float32),
        pltpu.VMEM_SHARED((N, width), jnp.float32),
    ]
    if with_deg:
        scratch.append(pltpu.VMEM_SHARED((N, 16), jnp.float32))
    scratch.append([pltpu.SemaphoreType.DMA for _ in range(NB)])
    scratch.append([pltpu.SemaphoreType.DMA for _ in range(NB)])

    def body(*refs):
        if with_deg:
            (x_hbm, a_hbm, w_hbm, out_hbm, deg_hbm,
             idx_s, idx_d, wq, rows, wv, acc, dacc, gsem, isem) = refs
        else:
            (x_hbm, a_hbm, w_hbm, out_hbm,
             idx_s, idx_d, wq, rows, wv, acc, gsem, isem) = refs
        c = lax.axis_index("c")
        s = lax.axis_index("s")
        wid = s * NC + c
        r0 = s * RPT
        e0 = wid * EPW

        pltpu.sync_copy(a_hbm.at[0, pl.ds(e0, EPW)], idx_s)

        zv = jnp.zeros((16,), jnp.float32)

        def zrow(r, c2):
            for j in range(width // 16):
                rows[0][r, pl.ds(j * 16, 16)] = zv
            wv[r, pl.ds(0, 16)] = zv
            return c2
        lax.fori_loop(0, K, zrow, 0)

        for q in range(RPT // K):
            pltpu.sync_copy(rows[0], acc.at[pl.ds(r0 + q * K, K)])
            if with_deg:
                pltpu.sync_copy(wv, dacc.at[pl.ds(r0 + q * K, K)])
        rem = RPT % K
        if rem:
            pltpu.sync_copy(rows[0].at[pl.ds(0, rem)],
                            acc.at[pl.ds(r0 + (RPT // K) * K, rem)])
            if with_deg:
                pltpu.sync_copy(wv.at[pl.ds(0, rem)],
                                dacc.at[pl.ds(r0 + (RPT // K) * K, rem)])

        @pl.when(s == 0)
        def _():
            t0 = NS * RPT
            pltpu.sync_copy(rows[0].at[pl.ds(0, RTAIL)],
                            acc.at[pl.ds(t0, RTAIL)])
            if with_deg:
                pltpu.sync_copy(wv.at[pl.ds(0, RTAIL)],
                                dacc.at[pl.ds(t0, RTAIL)])
        plsc.subcore_barrier()

        def issue(t, b):
            base = e0 + t * K
            pltpu.async_copy(a_hbm.at[1, pl.ds(base, K)], idx_d[b], isem[b])
            pltpu.async_copy(w_hbm.at[pl.ds(base, K)], wq[b], isem[b])
            pltpu.async_copy(x_hbm.at[idx_s.at[pl.ds(t * K, K)]], rows[b],
                             gsem[b])

        for b in range(NB):
            issue(b, b)

        def process(t, b, do_issue):
            pltpu.make_async_copy(a_hbm.at[1, pl.ds(e0, K)], idx_d[b],
                                  isem[b]).wait()
            pltpu.make_async_copy(w_hbm.at[pl.ds(e0, K)], wq[b],
                                  isem[b]).wait()
            pltpu.make_async_copy(x_hbm.at[pl.ds(0, K)], rows[b],
                                  gsem[b]).wait()

            def grp(g, c2):
                sp = [plsc.load_gather(
                    wq[b], [jnp.full((16,), g * 16 + i, jnp.int32)])
                    for i in range(16)]
                for i in range(16):
                    e = g * 16 + i
                    ws = sp[i]
                    if with_deg:
                        wv[e, pl.ds(0, 16)] = ws
                    for j in range(width // 16):
                        rows[b][e, pl.ds(j * 16, 16)] = (
                            rows[b][e, pl.ds(j * 16, 16)] * ws)
                return c2
            lax.fori_loop(0, K // 16, grp, 0)

            pltpu.sync_copy(rows[b], acc.at[idx_d[b]], add=True)
            if with_deg:
                pltpu.sync_copy(wv, dacc.at[idx_d[b]], add=True)

            if do_issue:
                @pl.when(t + NB < NCHUNK)
                def _():
                    issue(t + NB, b)

        def ring(t2, carry):
            for b in range(NB):
                process(t2 * NB + b, b, True)
            return carry
        lax.fori_loop(0, NCHUNK // NB, ring, 0)
        for r in range(NCHUNK % NB):
            process((NCHUNK // NB) * NB + r, r, False)

        plsc.subcore_barrier()
        pltpu.sync_copy(acc.at[pl.ds(r0, RPT)], out_hbm.at[c, pl.ds(r0, RPT)])
        if with_deg:
            pltpu.sync_copy(dacc.at[pl.ds(r0, RPT)],
                            deg_hbm.at[c, pl.ds(r0, RPT)])

        @pl.when(s == 0)
        def _():
            t0 = NS * RPT
            pltpu.sync_copy(acc.at[pl.ds(t0, RTAIL)],
                            out_hbm.at[c, pl.ds(t0, RTAIL)])
            if with_deg:
                pltpu.sync_copy(dacc.at[pl.ds(t0, RTAIL)],
                                deg_hbm.at[c, pl.ds(t0, RTAIL)])

    return pl.kernel(body, out_type=tuple(out_type), mesh=mesh,
                     scratch_types=tuple(scratch),
                     compiler_params=pltpu.CompilerParams(
                         use_tc_tiling_on_sc=False,
                         needs_layout_passes=False))


_sc_agg_l1 = _make_sc_agg(D, True)
_sc_agg_l2 = _make_sc_agg(C, False)


BROWS = 1000


def _tc1_body(p, d, x, wl1, wr1, b1, wl2, wr2, y2, r2):
    deg = d[0][:, 0:1] + d[1][:, 0:1]
    agg = (p[0] + p[1]) / jnp.clip(deg, 1e-12, None)
    out = (jnp.dot(agg, wl1[...], preferred_element_type=jnp.float32)
           + jnp.dot(x[...], wr1[...], preferred_element_type=jnp.float32)
           + b1[...])
    nrm = jnp.sqrt(jnp.sum(out * out, axis=1, keepdims=True))
    h = jnp.maximum(out / jnp.clip(nrm, 1e-12, None), 0.0)
    y2[...] = jnp.dot(h, wl2[...], preferred_element_type=jnp.float32)
    r2[...] = jnp.dot(h, wr2[...], preferred_element_type=jnp.float32)


_tc1 = pl.pallas_call(
    _tc1_body,
    grid=(N // BROWS,),
    in_specs=[
        pl.BlockSpec((2, BROWS, D), lambda i: (0, i, 0)),
        pl.BlockSpec((2, BROWS, 16), lambda i: (0, i, 0)),
        pl.BlockSpec((BROWS, D), lambda i: (i, 0)),
        pl.BlockSpec((D, H), lambda i: (0, 0)),
        pl.BlockSpec((D, H), lambda i: (0, 0)),
        pl.BlockSpec((1, H), lambda i: (0, 0)),
        pl.BlockSpec((H, C), lambda i: (0, 0)),
        pl.BlockSpec((H, C), lambda i: (0, 0)),
    ],
    out_specs=[
        pl.BlockSpec((BROWS, C), lambda i: (i, 0)),
        pl.BlockSpec((BROWS, C), lambda i: (i, 0)),
    ],
    out_shape=[
        jax.ShapeDtypeStruct((N, C), jnp.float32),
        jax.ShapeDtypeStruct((N, C), jnp.float32),
    ],
)


def _tc2_body(q, d, r2, b2, out):
    deg = d[0][:, 0:1] + d[1][:, 0:1]
    z = (q[0] + q[1]) / jnp.clip(deg, 1e-12, None) + r2[...] + b2[...]
    m = jnp.max(z, axis=1, keepdims=True)
    zs = z - m
    out[...] = zs - jnp.log(jnp.sum(jnp.exp(zs), axis=1, keepdims=True))


_tc2 = pl.pallas_call(
    _tc2_body,
    grid=(N // BROWS,),
    in_specs=[
        pl.BlockSpec((2, BROWS, C), lambda i: (0, i, 0)),
        pl.BlockSpec((2, BROWS, 16), lambda i: (0, i, 0)),
        pl.BlockSpec((BROWS, C), lambda i: (i, 0)),
        pl.BlockSpec((1, C), lambda i: (0, 0)),
    ],
    out_specs=pl.BlockSpec((BROWS, C), lambda i: (i, 0)),
    out_shape=jax.ShapeDtypeStruct((N, C), jnp.float32),
)


def kernel(X, A, W, Wl1, Wr1, b1, Wl2, Wr2, b2):
    P, DEG = _sc_agg_l1(X, A, W)
    Y2, R2 = _tc1(P, DEG, X, Wl1.T, Wr1.T, b1[None, :], Wl2.T, Wr2.T)
    (Q,) = _sc_agg_l2(Y2, A, W)
    return _tc2(Q, DEG, R2, b2[None, :])


# speedup vs baseline: 1.6657x; 1.0214x over previous
import jax
import jax.numpy as jnp
from jax import lax
from jax.experimental import pallas as pl
from jax.experimental.pallas import tpu as pltpu
from jax.experimental.pallas import tpu_sc as plsc

N = 10000
E = 320000
D = 128
H = 128
C = 64

NC = 2
NS = 16
NW = NC * NS
EPW = E // NW
K = 80
NCHUNK = EPW // K
RPT = 624
RTAIL = N - NS * RPT

def _make_sc_agg(width, with_deg):
    mesh = plsc.VectorSubcoreMesh(core_axis_name="c", subcore_axis_name="s",
                                  num_cores=NC, num_subcores=NS)
    out_type = [jax.ShapeDtypeStruct((NC, N, width), jnp.float32)]
    if with_deg:
        out_type.append(jax.ShapeDtypeStruct((NC, N, 16), jnp.float32))

    NB = 2 if width > 64 else 5
    scratch = [
        pltpu.VMEM((EPW,), jnp.int32),
        [pltpu.VMEM((K,), jnp.int32) for _ in range(NB)],
        [pltpu.VMEM((K,), jnp.float32) for _ in range(NB)],
        [pltpu.VMEM((K, width), jnp.float32) for _ in range(NB)],
        [pltpu.VMEM((K, 16), jnp.float32)
         for _ in range(NB if with_deg else 1)],
        pltpu.VMEM_SHARED((N, width), jnp.float32),
    ]
    if with_deg:
        scratch.append(pltpu.VMEM_SHARED((N, 16), jnp.float32))
    scratch.append([pltpu.SemaphoreType.DMA for _ in range(NB)])
    scratch.append([pltpu.SemaphoreType.DMA for _ in range(NB)])
    scratch.append([pltpu.SemaphoreType.DMA for _ in range(NB)])

    def body(*refs):
        if with_deg:
            (x_hbm, a_hbm, w_hbm, out_hbm, deg_hbm,
             idx_s, idx_d, wq, rows, wv, acc, dacc, gsem, isem, ssem) = refs
        else:
            (x_hbm, a_hbm, w_hbm, out_hbm,
             idx_s, idx_d, wq, rows, wv, acc, gsem, isem, ssem) = refs
        c = lax.axis_index("c")
        s = lax.axis_index("s")
        wid = s * NC + c
        r0 = s * RPT
        e0 = wid * EPW

        pltpu.sync_copy(a_hbm.at[0, pl.ds(e0, EPW)], idx_s)

        zv = jnp.zeros((16,), jnp.float32)

        def zrow(r, c2):
            for j in range(width // 16):
                rows[0][r, pl.ds(j * 16, 16)] = zv
            wv[0][r, pl.ds(0, 16)] = zv
            return c2
        lax.fori_loop(0, K, zrow, 0)

        for q in range(RPT // K):
            pltpu.sync_copy(rows[0], acc.at[pl.ds(r0 + q * K, K)])
            if with_deg:
                pltpu.sync_copy(wv[0], dacc.at[pl.ds(r0 + q * K, K)])
        rem = RPT % K
        if rem:
            pltpu.sync_copy(rows[0].at[pl.ds(0, rem)],
                            acc.at[pl.ds(r0 + (RPT // K) * K, rem)])
            if with_deg:
                pltpu.sync_copy(wv[0].at[pl.ds(0, rem)],
                                dacc.at[pl.ds(r0 + (RPT // K) * K, rem)])

        @pl.when(s == 0)
        def _():
            t0 = NS * RPT
            pltpu.sync_copy(rows[0].at[pl.ds(0, RTAIL)],
                            acc.at[pl.ds(t0, RTAIL)])
            if with_deg:
                pltpu.sync_copy(wv[0].at[pl.ds(0, RTAIL)],
                                dacc.at[pl.ds(t0, RTAIL)])
        plsc.subcore_barrier()

        def issue(t, b):
            base = e0 + t * K
            pltpu.async_copy(a_hbm.at[1, pl.ds(base, K)], idx_d[b], isem[b])
            pltpu.async_copy(w_hbm.at[pl.ds(base, K)], wq[b], isem[b])
            pltpu.async_copy(x_hbm.at[idx_s.at[pl.ds(t * K, K)]], rows[b],
                             gsem[b])

        for b in range(NB):
            issue(b, b)

        def wait_scatter(bq):
            pltpu.make_async_copy(x_hbm.at[pl.ds(0, K)], rows[bq],
                                  ssem[bq]).wait()
            if with_deg:
                pltpu.make_async_copy(deg_hbm.at[0, pl.ds(0, K)], wv[bq],
                                      ssem[bq]).wait()

        def process(t, b, do_issue):
            bp = (b - 1) % NB

            @pl.when(t >= 1)
            def _():
                wait_scatter(bp)
                if do_issue:
                    @pl.when(t - 1 + NB < NCHUNK)
                    def _():
                        issue(t - 1 + NB, bp)

            pltpu.make_async_copy(a_hbm.at[1, pl.ds(e0, K)], idx_d[b],
                                  isem[b]).wait()
            pltpu.make_async_copy(w_hbm.at[pl.ds(e0, K)], wq[b],
                                  isem[b]).wait()
            pltpu.make_async_copy(x_hbm.at[pl.ds(0, K)], rows[b],
                                  gsem[b]).wait()

            def grp(g, c2):
                sp = [plsc.load_gather(
                    wq[b], [jnp.full((16,), g * 16 + i, jnp.int32)])
                    for i in range(16)]
                for i in range(16):
                    e = g * 16 + i
                    ws = sp[i]
                    if with_deg:
                        wv[b][e, pl.ds(0, 16)] = ws
                    for j in range(width // 16):
                        rows[b][e, pl.ds(j * 16, 16)] = (
                            rows[b][e, pl.ds(j * 16, 16)] * ws)
                return c2
            lax.fori_loop(0, K // 16, grp, 0)

            pltpu.async_copy(rows[b], acc.at[idx_d[b]], ssem[b], add=True)
            if with_deg:
                pltpu.async_copy(wv[b], dacc.at[idx_d[b]], ssem[b], add=True)

        def ring(t2, carry):
            for b in range(NB):
                process(t2 * NB + b, b, True)
            return carry
        lax.fori_loop(0, NCHUNK // NB, ring, 0)
        for r in range(NCHUNK % NB):
            process((NCHUNK // NB) * NB + r, r, True)
        wait_scatter((NCHUNK - 1) % NB)

        plsc.subcore_barrier()
        pltpu.sync_copy(acc.at[pl.ds(r0, RPT)], out_hbm.at[c, pl.ds(r0, RPT)])
        if with_deg:
            pltpu.sync_copy(dacc.at[pl.ds(r0, RPT)],
                            deg_hbm.at[c, pl.ds(r0, RPT)])

        @pl.when(s == 0)
        def _():
            t0 = NS * RPT
            pltpu.sync_copy(acc.at[pl.ds(t0, RTAIL)],
                            out_hbm.at[c, pl.ds(t0, RTAIL)])
            if with_deg:
                pltpu.sync_copy(dacc.at[pl.ds(t0, RTAIL)],
                                deg_hbm.at[c, pl.ds(t0, RTAIL)])

    return pl.kernel(body, out_type=tuple(out_type), mesh=mesh,
                     scratch_types=tuple(scratch),
                     compiler_params=pltpu.CompilerParams(
                         use_tc_tiling_on_sc=False,
                         needs_layout_passes=False))


_sc_agg_l1 = _make_sc_agg(D, True)
_sc_agg_l2 = _make_sc_agg(C, False)


BROWS = 1000


def _dot_t(a, w):
    return lax.dot_general(a, w, (((1,), (1,)), ((), ())),
                           preferred_element_type=jnp.float32)


def _tc1_body(p, d, x, wl1, wr1, b1, wl2, wr2, y2, r2):
    deg = d[0][:, 0:1] + d[1][:, 0:1]
    agg = (p[0] + p[1]) / jnp.clip(deg, 1e-12, None)
    out = _dot_t(agg, wl1[...]) + _dot_t(x[...], wr1[...]) + b1[...]
    nrm = jnp.sqrt(jnp.sum(out * out, axis=1, keepdims=True))
    h = jnp.maximum(out / jnp.clip(nrm, 1e-12, None), 0.0)
    y2[...] = _dot_t(h, wl2[...])
    r2[...] = _dot_t(h, wr2[...])


_tc1 = pl.pallas_call(
    _tc1_body,
    grid=(N // BROWS,),
    in_specs=[
        pl.BlockSpec((2, BROWS, D), lambda i: (0, i, 0)),
        pl.BlockSpec((2, BROWS, 16), lambda i: (0, i, 0)),
        pl.BlockSpec((BROWS, D), lambda i: (i, 0)),
        pl.BlockSpec((H, D), lambda i: (0, 0)),
        pl.BlockSpec((H, D), lambda i: (0, 0)),
        pl.BlockSpec((1, H), lambda i: (0, 0)),
        pl.BlockSpec((C, H), lambda i: (0, 0)),
        pl.BlockSpec((C, H), lambda i: (0, 0)),
    ],
    out_specs=[
        pl.BlockSpec((BROWS, C), lambda i: (i, 0)),
        pl.BlockSpec((BROWS, C), lambda i: (i, 0)),
    ],
    out_shape=[
        jax.ShapeDtypeStruct((N, C), jnp.float32),
        jax.ShapeDtypeStruct((N, C), jnp.float32),
    ],
)


def _tc2_body(q, d, r2, b2, out):
    deg = d[0][:, 0:1] + d[1][:, 0:1]
    z = (q[0] + q[1]) / jnp.clip(deg, 1e-12, None) + r2[...] + b2[...]
    m = jnp.max(z, axis=1, keepdims=True)
    zs = z - m
    out[...] = zs - jnp.log(jnp.sum(jnp.exp(zs), axis=1, keepdims=True))


_tc2 = pl.pallas_call(
    _tc2_body,
    grid=(N // BROWS,),
    in_specs=[
        pl.BlockSpec((2, BROWS, C), lambda i: (0, i, 0)),
        pl.BlockSpec((2, BROWS, 16), lambda i: (0, i, 0)),
        pl.BlockSpec((BROWS, C), lambda i: (i, 0)),
        pl.BlockSpec((1, C), lambda i: (0, 0)),
    ],
    out_specs=pl.BlockSpec((BROWS, C), lambda i: (i, 0)),
    out_shape=jax.ShapeDtypeStruct((N, C), jnp.float32),
)


def kernel(X, A, W, Wl1, Wr1, b1, Wl2, Wr2, b2):
    P, DEG = _sc_agg_l1(X, A, W)
    Y2, R2 = _tc1(P, DEG, X, Wl1, Wr1, b1[None, :], Wl2, Wr2)
    (Q,) = _sc_agg_l2(Y2, A, W)
    return _tc2(Q, DEG, R2, b2[None, :])
